# baseline (speedup 1.0000x reference)
import functools

import jax
import jax.numpy as jnp
from jax import lax
from jax.experimental import pallas as pl
from jax.experimental.pallas import tpu as pltpu
from jax.experimental.pallas import tpu_sc as plsc

_N = 100000
_MAX_RANK = 20
_NW = 32
_CHUNK = 3136
_PAD = _NW * _CHUNK
_CELL = 224
_NCELL = _PAD // _CELL
_CPW = _CHUNK // _CELL
_VPC = _CELL // 16
_KEY_MUL = 1 << 17
_NEGF = -3.0e38


def _sc_cellmax_call(logits_p, ranks_p):
    mesh = plsc.VectorSubcoreMesh(core_axis_name="c", subcore_axis_name="s")

    @functools.partial(
        pl.kernel,
        mesh=mesh,
        out_type=[
            jax.ShapeDtypeStruct((_NW, 16), jnp.float32),
            jax.ShapeDtypeStruct((_NW, 16), jnp.float32),
        ],
        scratch_types=[
            pltpu.VMEM((_CHUNK,), jnp.float32),
            pltpu.VMEM((_CHUNK,), jnp.int32),
            pltpu.VMEM((16,), jnp.float32),
            pltpu.VMEM((16,), jnp.float32),
        ],
    )
    def sc_cellmax(logits_hbm, ranks_hbm, outk, outl, lg_v, rk_v, kbuf, lbuf):
        wid = lax.axis_index("s") * 2 + lax.axis_index("c")
        base = wid * _CHUNK
        pltpu.sync_copy(logits_hbm.at[pl.ds(base, _CHUNK)], lg_v)
        pltpu.sync_copy(ranks_hbm.at[pl.ds(base, _CHUNK)], rk_v)
        lane = lax.iota(jnp.int32, 16)
        perms = [lane ^ 8, lane ^ 4, lane ^ 2, lane ^ 1]

        kout = jnp.full((16,), _NEGF, jnp.float32)
        lout = jnp.full((16,), _NEGF, jnp.float32)
        for c in range(_CPW):
            kmax = None
            lmax = None
            for v in range(_VPC):
                off = c * _CELL + v * 16
                k = (rk_v[pl.ds(off, 16)] * _KEY_MUL + (base + off + lane)).astype(
                    jnp.float32
                )
                l = lg_v[pl.ds(off, 16)]
                kmax = k if kmax is None else jnp.maximum(kmax, k)
                lmax = l if lmax is None else jnp.maximum(lmax, l)
            for perm in perms:
                kmax = jnp.maximum(kmax, kmax.at[perm].get(mode="promise_in_bounds"))
                lmax = jnp.maximum(lmax, lmax.at[perm].get(mode="promise_in_bounds"))
            sel = lane == c
            kout = jnp.where(sel, kmax, kout)
            lout = jnp.where(sel, lmax, lout)

        kbuf[...] = kout
        lbuf[...] = lout
        pltpu.sync_copy(kbuf, outk.at[wid])
        pltpu.sync_copy(lbuf, outl.at[wid])

    return sc_cellmax(logits_p, ranks_p)


def _tc_final_body(cellk_ref, celll_ref, ranks_ref, logits_ref, out_ref, skeys, slogits):
    ck = cellk_ref[...]
    cl = celll_ref[...]

    row = lax.broadcasted_iota(jnp.int32, (4, 128), 0)
    col = lax.broadcasted_iota(jnp.int32, (4, 128), 1)
    pos = row * 128 + col
    lane = pos % 16
    cid = (pos // 16) * _CPW + lane
    valid = lane < _CPW
    ckm = jnp.where(valid, ck, _NEGF)

    sel_ids = []
    selmask = jnp.zeros((4, 128), jnp.bool_)
    cur = ckm
    for _ in range(_MAX_RANK):
        mk = jnp.max(cur)
        hit = cur == mk
        sel_ids.append(jnp.max(jnp.where(hit, cid, -1)))
        selmask = selmask | hit
        cur = jnp.where(hit, _NEGF, cur)

    ccol = lax.broadcasted_iota(jnp.int32, (1, _CELL), 1)
    for s in range(_MAX_RANK):
        sid = sel_ids[s]
        r = ranks_ref[pl.ds(sid, 1), :]
        lg = logits_ref[pl.ds(sid, 1), :]
        key = (r * _KEY_MUL + (sid * _CELL + ccol)).astype(jnp.float32)
        skeys[pl.ds(s, 1), :] = key
        slogits[pl.ds(s, 1), :] = lg

    K = skeys[...]
    L = slogits[...]
    sel_k = []
    sel_l = []
    cur = K
    for _ in range(_MAX_RANK):
        mk = jnp.max(cur)
        hit = cur == mk
        lg = jnp.max(jnp.where(hit, L, _NEGF))
        sel_k.append(mk)
        sel_l.append(lg)
        cur = jnp.where(hit, _NEGF, cur)

    n_labels = jnp.floor(sel_k[0] * (1.0 / _KEY_MUL)).astype(jnp.int32)

    thr = jnp.float32(0.0)
    low_lg = jnp.float32(0.0)
    for j in range(_MAX_RANK):
        pick = n_labels - 1 == j
        thr = jnp.where(pick, sel_k[j], thr)
        low_lg = jnp.where(pick, sel_l[j], low_lg)

    rem_lg = jnp.maximum(
        jnp.max(jnp.where(K < thr, L, _NEGF)),
        jnp.max(jnp.where(selmask, _NEGF, cl)),
    )

    probs = [jax.nn.sigmoid(lg) for lg in sel_l]
    loss1 = jnp.maximum(jax.nn.sigmoid(low_lg) - jax.nn.sigmoid(rem_lg), 0.0)

    correct = jnp.int32(0)
    for i in range(_MAX_RANK - 1):
        for j in range(i + 1, _MAX_RANK):
            c = (j < n_labels) & (probs[i] > probs[j])
            correct = correct + c.astype(jnp.int32)
    total = n_labels * (n_labels - 1) // 2
    loss2 = jnp.where(
        total > 0,
        1.0 - correct.astype(jnp.float32) / jnp.maximum(total, 1).astype(jnp.float32),
        jnp.float32(0.0),
    )
    out = jnp.where(n_labels != 0, loss1 + loss2, jnp.float32(0.0))
    out_ref[...] = jnp.broadcast_to(out, (1, 1))


def kernel(logits, ranks):
    logits_p = jnp.full((_PAD,), _NEGF, jnp.float32).at[:_N].set(logits)
    ranks_p = jnp.full((_PAD,), -1, jnp.int32).at[:_N].set(ranks)
    cellk, celll = _sc_cellmax_call(logits_p, ranks_p)
    out = pl.pallas_call(
        _tc_final_body,
        out_shape=jax.ShapeDtypeStruct((1, 1), jnp.float32),
        scratch_shapes=[
            pltpu.VMEM((_MAX_RANK, _CELL), jnp.float32),
            pltpu.VMEM((_MAX_RANK, _CELL), jnp.float32),
        ],
    )(
        cellk.reshape(4, 128),
        celll.reshape(4, 128),
        ranks_p.reshape(_NCELL, _CELL),
        logits_p.reshape(_NCELL, _CELL),
    )
    return out[0, 0]


# speedup vs baseline: 1.0981x; 1.0981x over previous
import functools

import jax
import jax.numpy as jnp
from jax import lax
from jax.experimental import pallas as pl
from jax.experimental.pallas import tpu as pltpu
from jax.experimental.pallas import tpu_sc as plsc

_N = 100000
_MAX_RANK = 20
_NW = 32
_CHUNK = 3584
_PAD = _NW * _CHUNK
_CELL = 224
_NCELL = _PAD // _CELL
_CPW = _CHUNK // _CELL
_VPC = _CELL // 16
_KEY_MUL = 1 << 17
_NEGF = -3.0e38


def _sc_cellmax_call(logits_p, ranks_p):
    mesh = plsc.VectorSubcoreMesh(core_axis_name="c", subcore_axis_name="s")

    @functools.partial(
        pl.kernel,
        mesh=mesh,
        out_type=[
            jax.ShapeDtypeStruct((_NW, 16), jnp.float32),
            jax.ShapeDtypeStruct((_NW, 16), jnp.float32),
        ],
        scratch_types=[
            pltpu.VMEM((_CHUNK,), jnp.float32),
            pltpu.VMEM((_CHUNK,), jnp.int32),
            pltpu.VMEM((16,), jnp.float32),
            pltpu.VMEM((16,), jnp.float32),
        ],
    )
    def sc_cellmax(logits_hbm, ranks_hbm, outk, outl, lg_v, rk_v, kbuf, lbuf):
        wid = lax.axis_index("s") * 2 + lax.axis_index("c")
        base = wid * _CHUNK
        pltpu.sync_copy(logits_hbm.at[pl.ds(base, _CHUNK)], lg_v)
        pltpu.sync_copy(ranks_hbm.at[pl.ds(base, _CHUNK)], rk_v)
        lane = lax.iota(jnp.int32, 16)
        perms = [lane ^ 8, lane ^ 4, lane ^ 2, lane ^ 1]

        kout = jnp.full((16,), _NEGF, jnp.float32)
        lout = jnp.full((16,), _NEGF, jnp.float32)
        for c in range(_CPW):
            kmax = None
            lmax = None
            for v in range(_VPC):
                off = c * _CELL + v * 16
                k = (rk_v[pl.ds(off, 16)] * _KEY_MUL + (base + off + lane)).astype(
                    jnp.float32
                )
                l = lg_v[pl.ds(off, 16)]
                kmax = k if kmax is None else jnp.maximum(kmax, k)
                lmax = l if lmax is None else jnp.maximum(lmax, l)
            for perm in perms:
                kmax = jnp.maximum(kmax, kmax.at[perm].get(mode="promise_in_bounds"))
                lmax = jnp.maximum(lmax, lmax.at[perm].get(mode="promise_in_bounds"))
            sel = lane == c
            kout = jnp.where(sel, kmax, kout)
            lout = jnp.where(sel, lmax, lout)

        kbuf[...] = kout
        lbuf[...] = lout
        pltpu.sync_copy(kbuf, outk.at[wid])
        pltpu.sync_copy(lbuf, outl.at[wid])

    return sc_cellmax(logits_p, ranks_p)


def _tc_final_body(ckcol_ref, ckrow_ref, clrow_ref, ranksf_ref, logits_ref, out_ref):
    ck_col = ckcol_ref[...]
    ck_row = ckrow_ref[...]
    cl_row = clrow_ref[...]

    C = jnp.where(jnp.broadcast_to(ck_col, (_NCELL, _NCELL))
                  > jnp.broadcast_to(ck_row, (_NCELL, _NCELL)), 1.0, 0.0)
    ones_row = jnp.full((1, _NCELL), 1.0, jnp.float32)
    rank_row = jax.lax.dot_general(
        ones_row, C.astype(jnp.float32), (((1,), (0,)), ((), ())),
        preferred_element_type=jnp.float32,
    )
    srows = lax.broadcasted_iota(jnp.int32, (_MAX_RANK, _NCELL), 0).astype(jnp.float32)
    S = jnp.where(jnp.broadcast_to(rank_row, (_MAX_RANK, _NCELL)) == srows, 1.0, 0.0)
    D = ranksf_ref[...]
    L = logits_ref[...]
    g_rank = jax.lax.dot_general(
        S, D, (((1,), (0,)), ((), ())), preferred_element_type=jnp.float32
    )
    g_log = jax.lax.dot_general(
        S, L, (((1,), (0,)), ((), ())), preferred_element_type=jnp.float32
    )
    cidv = lax.broadcasted_iota(jnp.int32, (_NCELL, 1), 0).astype(jnp.float32)
    g_cid = jax.lax.dot_general(
        S, cidv, (((1,), (0,)), ((), ())), preferred_element_type=jnp.float32
    )

    gcol = lax.broadcasted_iota(jnp.int32, (_MAX_RANK, _CELL), 1).astype(jnp.float32)
    K = g_rank * float(_KEY_MUL) + (g_cid * float(_CELL) + gcol)

    sel_k = []
    cur = K
    for _ in range(_MAX_RANK):
        mk = jnp.max(cur)
        sel_k.append(mk)
        cur = jnp.where(cur == mk, _NEGF, cur)
    sel_l = [jnp.max(jnp.where(K == k, g_log, _NEGF)) for k in sel_k]

    n_labels = jnp.floor(sel_k[0] * (1.0 / _KEY_MUL)).astype(jnp.int32)
    n_lab_f = n_labels.astype(jnp.float32)

    lane20 = lax.broadcasted_iota(jnp.int32, (1, _MAX_RANK), 1)
    skv = jnp.full((1, _MAX_RANK), _NEGF, jnp.float32)
    slv = jnp.full((1, _MAX_RANK), _NEGF, jnp.float32)
    pcol = jnp.full((_MAX_RANK, 1), _NEGF, jnp.float32)
    sub20 = lax.broadcasted_iota(jnp.int32, (_MAX_RANK, 1), 0)
    for j in range(_MAX_RANK):
        skv = jnp.where(lane20 == j, sel_k[j], skv)
        slv = jnp.where(lane20 == j, sel_l[j], slv)
        pcol = jnp.where(sub20 == j, sel_l[j], pcol)

    pick = lane20 == n_labels - 1
    thr = jnp.max(jnp.where(pick, skv, _NEGF))
    low_lg = jnp.max(jnp.where(pick, slv, _NEGF))

    rem_lg = jnp.maximum(
        jnp.max(jnp.where(K < thr, g_log, _NEGF)),
        jnp.max(jnp.where(rank_row < 20.0, _NEGF, cl_row)),
    )

    pi = jax.nn.sigmoid(jnp.broadcast_to(pcol, (_MAX_RANK, _MAX_RANK)))
    pj = jax.nn.sigmoid(jnp.broadcast_to(slv, (_MAX_RANK, _MAX_RANK)))
    i_idx = lax.broadcasted_iota(jnp.int32, (_MAX_RANK, _MAX_RANK), 0)
    j_idx = lax.broadcasted_iota(jnp.int32, (_MAX_RANK, _MAX_RANK), 1)
    pair_ok = (i_idx < j_idx) & (j_idx < n_labels) & (pi > pj)
    correct = jnp.sum(jnp.where(pair_ok, 1.0, 0.0))

    loss1 = jnp.maximum(jax.nn.sigmoid(low_lg) - jax.nn.sigmoid(rem_lg), 0.0)
    total = (n_lab_f * (n_lab_f - 1.0)) * 0.5
    loss2 = jnp.where(total > 0.0, 1.0 - correct / jnp.maximum(total, 1.0), 0.0)
    out = jnp.where(n_labels != 0, loss1 + loss2, jnp.float32(0.0))
    out_ref[...] = jnp.broadcast_to(out, (1, 1))


def kernel(logits, ranks):
    logits_p = jnp.full((_PAD,), _NEGF, jnp.float32).at[:_N].set(logits)
    ranks_p = jnp.full((_PAD,), -1, jnp.int32).at[:_N].set(ranks)
    cellk, celll = _sc_cellmax_call(logits_p, ranks_p)
    out = pl.pallas_call(
        _tc_final_body,
        out_shape=jax.ShapeDtypeStruct((1, 1), jnp.float32),
    )(
        cellk.reshape(_NCELL, 1),
        cellk.reshape(1, _NCELL),
        celll.reshape(1, _NCELL),
        ranks_p.astype(jnp.float32).reshape(_NCELL, _CELL),
        logits_p.reshape(_NCELL, _CELL),
    )
    return out[0, 0]
